# baseline (speedup 1.0000x reference)
import functools

import jax
import jax.numpy as jnp
from jax import lax
from jax.experimental import pallas as pl
from jax.experimental.pallas import tpu as pltpu
from jax.experimental.pallas import tpu_sc as plsc

_NEG_RATIO = 3.0
_BAL_SCALE = 5.0
_EPS = 1e-09
_L1_SCALE = 10.0
_DICE_EPS = 0.001

_B = 8
_ROWS_PER_B = 400
_LANES = 1024
_RB = 80
_JB = _ROWS_PER_B // _RB
_ROWS = _B * _ROWS_PER_B
_N = _ROWS * _LANES

_CH = 16
_NCH = _ROWS // _CH

_BIG = 3.0e38

_NB = 4096
_VMAX = 27.632
_BIN_SCALE = _NB / _VMAX
_BIN_W = _VMAX / _NB
_LN = 16
_WRK = 32
_PER_W = _N // _WRK
_CHUNK = 4096
_NCHUNK = _PER_W // _CHUNK
_REFINE_ITERS = 4


def _stream_body(pr_ref, gt_ref, neg_ref, sum_ref, mm_ref, dice_ref):
    b = pl.program_id(0)
    j = pl.program_id(1)

    @pl.when(jnp.logical_and(b == 0, j == 0))
    def _init():
        sum_ref[...] = jnp.zeros((8, _LANES), jnp.float32)
        dice_ref[...] = jnp.zeros((8, _B, _LANES), jnp.float32)
        mm_ref[...] = jnp.zeros((8, _LANES), jnp.float32)
        mm_ref[0:1, :] = jnp.full((1, _LANES), _BIG, jnp.float32)
        mm_ref[1:2, :] = jnp.full((1, _LANES), -_BIG, jnp.float32)

    binary = pr_ref[0, 0]
    thresh_binary = pr_ref[0, 1]
    thresh = pr_ref[0, 2]
    gt = gt_ref[0, 0]
    mask = gt_ref[0, 1]
    thresh_map = gt_ref[0, 2]
    thresh_mask = gt_ref[0, 3]

    p = jnp.clip(binary, 1e-12, 1.0 - 1e-12)
    logp = jnp.clip(jnp.log(p), -100.0, None)
    log1mp = jnp.clip(jnp.log(1.0 - p), -100.0, None)
    loss = -(gt * logp + (1.0 - gt) * log1mp)

    pos_m = gt * mask
    neg_m = (1.0 - gt) * mask
    negl = loss * neg_m

    neg_ref[0] = negl

    def _acc_sum(row, val2d):
        sum_ref[row:row + 1, :] += jnp.sum(val2d, axis=0, keepdims=True)

    _acc_sum(0, jnp.abs(thresh - thresh_map) * thresh_mask)
    _acc_sum(1, thresh_mask)
    _acc_sum(2, pos_m)
    _acc_sum(3, neg_m)
    _acc_sum(4, loss * pos_m)

    mm_ref[0:1, :] = jnp.minimum(mm_ref[0:1, :],
                                 jnp.min(loss, axis=0, keepdims=True))
    mm_ref[1:2, :] = jnp.maximum(mm_ref[1:2, :],
                                 jnp.max(loss, axis=0, keepdims=True))

    ptm = thresh_binary * gt * mask
    ppm = thresh_binary * thresh_binary * mask
    ttm = gt * gt * mask

    def _acc_dice(q, val2d):
        dice_ref[q, pl.ds(b, 1), :] += jnp.sum(val2d, axis=0, keepdims=True)

    _acc_dice(0, ptm)
    _acc_dice(1, ptm * loss)
    _acc_dice(2, ppm)
    _acc_dice(3, ppm * loss)
    _acc_dice(4, ttm)
    _acc_dice(5, ttm * loss)


def _stream(y_pr4, y_gt4):
    return pl.pallas_call(
        _stream_body,
        grid=(_B, _JB),
        in_specs=[
            pl.BlockSpec((1, 3, _RB, _LANES), lambda b, j: (b, 0, j, 0)),
            pl.BlockSpec((1, 4, _RB, _LANES), lambda b, j: (b, 0, j, 0)),
        ],
        out_specs=[
            pl.BlockSpec((1, _RB, _LANES), lambda b, j: (b, j, 0)),
            pl.BlockSpec((8, _LANES), lambda b, j: (0, 0)),
            pl.BlockSpec((8, _LANES), lambda b, j: (0, 0)),
            pl.BlockSpec((8, _B, _LANES), lambda b, j: (0, 0, 0)),
        ],
        out_shape=[
            jax.ShapeDtypeStruct((_B, _ROWS_PER_B, _LANES), jnp.float32),
            jax.ShapeDtypeStruct((8, _LANES), jnp.float32),
            jax.ShapeDtypeStruct((8, _LANES), jnp.float32),
            jax.ShapeDtypeStruct((8, _B, _LANES), jnp.float32),
        ],
        compiler_params=pltpu.CompilerParams(
            dimension_semantics=("arbitrary", "arbitrary"),
        ),
    )(y_pr4, y_gt4)


@functools.lru_cache(maxsize=1)
def _build_sc_hist():
    @functools.partial(
        pl.kernel,
        out_type=jax.ShapeDtypeStruct((_WRK, _NB), jnp.float32),
        mesh=plsc.VectorSubcoreMesh(core_axis_name="c", subcore_axis_name="s"),
        scratch_types=[
            pltpu.VMEM((_CHUNK,), jnp.float32),
            pltpu.VMEM((_NB * _LN,), jnp.float32),
            pltpu.VMEM((_NB,), jnp.float32),
        ],
        compiler_params=pltpu.CompilerParams(needs_layout_passes=False),
    )
    def _sc_hist(neg_hbm, out_hbm, chunk_v, hist_v, merged_v):
        wid = lax.axis_index("s") * 2 + lax.axis_index("c")
        lane_base = lax.iota(jnp.int32, 16) * _NB
        ones = jnp.ones((16,), jnp.float32)
        zeros16 = jnp.zeros((16,), jnp.float32)

        def zero_body(i, carry):
            for u in range(8):
                hist_v[pl.ds(i * 128 + u * 16, 16)] = zeros16
            return carry

        lax.fori_loop(0, _NB * _LN // 128, zero_body, 0)

        base = wid * _PER_W
        for ci in range(_NCHUNK):
            pltpu.sync_copy(neg_hbm.at[pl.ds(base + ci * _CHUNK, _CHUNK)],
                            chunk_v)

            def body(i, carry):
                for u in range(4):
                    v = chunk_v[pl.ds(i * 64 + u * 16, 16)]
                    bn = jnp.clip((v * _BIN_SCALE).astype(jnp.int32),
                                  0, _NB - 1)
                    plsc.addupdate_scatter(hist_v, [lane_base + bn], ones)
                return carry

            lax.fori_loop(0, _CHUNK // 64, body, 0)

        def merge_body(i, carry):
            acc = zeros16
            for l in range(_LN):
                acc = acc + hist_v[pl.ds(l * _NB + i * 16, 16)]
            merged_v[pl.ds(i * 16, 16)] = acc
            return carry

        lax.fori_loop(0, _NB // 16, merge_body, 0)

        pltpu.sync_copy(merged_v, out_hbm.at[wid])

    return _sc_hist


def _fin_body(neg_ref, hist_ref, sum_ref, mm_ref, dice_ref, out_ref):
    l1_num = jnp.sum(sum_ref[0, :])
    l1_den = jnp.sum(sum_ref[1, :])
    pos_cnt = jnp.sum(sum_ref[2, :])
    neg_cnt_raw = jnp.sum(sum_ref[3, :])
    pos_loss_sum = jnp.sum(sum_ref[4, :])
    dmin = jnp.min(mm_ref[0, :])
    dmax = jnp.max(mm_ref[1, :])

    k_f = jnp.minimum(neg_cnt_raw, pos_cnt * _NEG_RATIO)
    k_i = k_f.astype(jnp.int32)
    k_if = k_i.astype(jnp.float32)

    hist_tot = jnp.sum(hist_ref[...], axis=0)
    bin2 = (lax.broadcasted_iota(jnp.int32, (4, _LANES), 0) * _LANES
            + lax.broadcasted_iota(jnp.int32, (4, _LANES), 1))

    def suffix_count(bidx):
        return jnp.sum(jnp.where(bin2 >= bidx, hist_tot, 0.0))

    def bin_bisect(_, carry):
        lo, hi = carry
        mid = (lo + hi) // 2
        pred = suffix_count(mid) >= k_if
        return (jnp.where(pred, mid, lo), jnp.where(pred, hi, mid))

    b_lo, b_hi = lax.fori_loop(0, 12, bin_bisect,
                               (jnp.int32(0), jnp.int32(_NB)))

    t_lo = b_lo.astype(jnp.float32) * _BIN_W
    t_hi = b_hi.astype(jnp.float32) * _BIN_W

    zed = jnp.zeros((_CH, _LANES), jnp.float32)

    def count_gt(t):
        def chunk(i, acc):
            blk = neg_ref[pl.ds(i * _CH, _CH), :]
            return acc + (blk > t).astype(jnp.float32)
        return jnp.sum(lax.fori_loop(0, _NCH, chunk, zed))

    def refine(_, carry):
        lo, hi = carry
        mid = 0.5 * (lo + hi)
        pred = count_gt(mid) > k_if
        return (jnp.where(pred, mid, lo), jnp.where(pred, hi, mid))

    _, t = lax.fori_loop(0, _REFINE_ITERS, refine, (t_lo, t_hi))

    def final_chunk(i, carry):
        c, s = carry
        blk = neg_ref[pl.ds(i * _CH, _CH), :]
        m = blk > t
        return (c + m.astype(jnp.float32), s + jnp.where(m, blk, 0.0))

    cnt_v, ssum_v = lax.fori_loop(0, _NCH, final_chunk, (zed, zed))
    cnt = jnp.sum(cnt_v)
    ssum = jnp.sum(ssum_v)
    topk_sum = ssum + (k_if - cnt) * t

    balanced = (pos_loss_sum + topk_sum) / (pos_cnt + k_f + _EPS)
    balanced = balanced * _BAL_SCALE

    a = 1.0 / (dmax - dmin)
    c0 = 1.0 - dmin * a
    dice_total = jnp.float32(0.0)
    for bb in range(_B):
        s_ptm = jnp.sum(dice_ref[0, bb, :])
        s_ptml = jnp.sum(dice_ref[1, bb, :])
        s_ppm = jnp.sum(dice_ref[2, bb, :])
        s_ppml = jnp.sum(dice_ref[3, bb, :])
        s_ttm = jnp.sum(dice_ref[4, bb, :])
        s_ttml = jnp.sum(dice_ref[5, bb, :])
        inter = a * s_ptml + c0 * s_ptm
        union = a * (s_ppml + s_ttml) + c0 * (s_ppm + s_ttm) + 2.0 * _DICE_EPS
        dice_total += 1.0 - 2.0 * inter / union
    dice = dice_total / _B

    l1 = jnp.where(l1_den > 0, l1_num / l1_den, jnp.float32(0.0)) * _L1_SCALE

    out_ref[0, 0] = l1 + balanced + dice


def _finalize(neg, hist3, sums, mm, dice):
    return pl.pallas_call(
        _fin_body,
        out_specs=pl.BlockSpec(memory_space=pltpu.SMEM),
        out_shape=jax.ShapeDtypeStruct((1, 1), jnp.float32),
    )(neg, hist3, sums, mm, dice)


def kernel(y_pr, y_gt):
    y_pr4 = y_pr.reshape(_B, 3, _ROWS_PER_B, _LANES)
    y_gt4 = y_gt.reshape(_B, 4, _ROWS_PER_B, _LANES)
    neg, sums, mm, dice = _stream(y_pr4, y_gt4)
    hist = _build_sc_hist()(neg.reshape(_N))
    out = _finalize(neg.reshape(_ROWS, _LANES),
                    hist.reshape(_WRK, 4, _LANES), sums, mm, dice)
    return out[0, 0]


# speedup vs baseline: 1.0596x; 1.0596x over previous
import functools

import jax
import jax.numpy as jnp
from jax import lax
from jax.experimental import pallas as pl
from jax.experimental.pallas import tpu as pltpu
from jax.experimental.pallas import tpu_sc as plsc

_NEG_RATIO = 3.0
_BAL_SCALE = 5.0
_EPS = 1e-09
_L1_SCALE = 10.0
_DICE_EPS = 0.001

_B = 8
_ROWS_PER_B = 400
_LANES = 1024
_RB = 80
_JB = _ROWS_PER_B // _RB
_ROWS = _B * _ROWS_PER_B
_N = _ROWS * _LANES

_CH = 16
_NCH = _ROWS // _CH

_BIG = 3.0e38

_NB = 4096
_VMAX = 27.632
_BIN_SCALE = _NB / _VMAX
_BIN_W = _VMAX / _NB
_LN = 16
_WRK = 32
_PER_W = _N // _WRK
_CHUNK = 4096
_NCHUNK = _PER_W // _CHUNK
_REFINE_ITERS = 4


def _stream_body(pr_ref, gt_ref, neg_ref, sum_ref, mm_ref, dice_ref):
    b = pl.program_id(0)
    j = pl.program_id(1)

    @pl.when(jnp.logical_and(b == 0, j == 0))
    def _init():
        sum_ref[...] = jnp.zeros((8, _LANES), jnp.float32)
        dice_ref[...] = jnp.zeros((8, _B, _LANES), jnp.float32)
        mm_ref[...] = jnp.zeros((8, _LANES), jnp.float32)
        mm_ref[0:1, :] = jnp.full((1, _LANES), _BIG, jnp.float32)
        mm_ref[1:2, :] = jnp.full((1, _LANES), -_BIG, jnp.float32)

    binary = pr_ref[0, 0]
    thresh_binary = pr_ref[0, 1]
    thresh = pr_ref[0, 2]
    gt = gt_ref[0, 0]
    mask = gt_ref[0, 1]
    thresh_map = gt_ref[0, 2]
    thresh_mask = gt_ref[0, 3]

    p = jnp.clip(binary, 1e-12, 1.0 - 1e-12)
    logp = jnp.clip(jnp.log(p), -100.0, None)
    log1mp = jnp.clip(jnp.log(1.0 - p), -100.0, None)
    loss = -(gt * logp + (1.0 - gt) * log1mp)

    pos_m = gt * mask
    neg_m = (1.0 - gt) * mask
    negl = loss * neg_m

    neg_ref[0] = negl

    def _acc_sum(row, val2d):
        sum_ref[row:row + 1, :] += jnp.sum(val2d, axis=0, keepdims=True)

    _acc_sum(0, jnp.abs(thresh - thresh_map) * thresh_mask)
    _acc_sum(1, thresh_mask)
    _acc_sum(2, pos_m)
    _acc_sum(3, neg_m)
    _acc_sum(4, loss * pos_m)

    mm_ref[0:1, :] = jnp.minimum(mm_ref[0:1, :],
                                 jnp.min(loss, axis=0, keepdims=True))
    mm_ref[1:2, :] = jnp.maximum(mm_ref[1:2, :],
                                 jnp.max(loss, axis=0, keepdims=True))

    ptm = thresh_binary * gt * mask
    ppm = thresh_binary * thresh_binary * mask
    ttm = gt * gt * mask

    def _acc_dice(q, val2d):
        dice_ref[q, pl.ds(b, 1), :] += jnp.sum(val2d, axis=0, keepdims=True)

    _acc_dice(0, ptm)
    _acc_dice(1, ptm * loss)
    _acc_dice(2, ppm)
    _acc_dice(3, ppm * loss)
    _acc_dice(4, ttm)
    _acc_dice(5, ttm * loss)


def _stream(y_pr4, y_gt4):
    return pl.pallas_call(
        _stream_body,
        grid=(_B, _JB),
        in_specs=[
            pl.BlockSpec((1, 3, _RB, _LANES), lambda b, j: (b, 0, j, 0)),
            pl.BlockSpec((1, 4, _RB, _LANES), lambda b, j: (b, 0, j, 0)),
        ],
        out_specs=[
            pl.BlockSpec((1, _RB, _LANES), lambda b, j: (b, j, 0)),
            pl.BlockSpec((8, _LANES), lambda b, j: (0, 0)),
            pl.BlockSpec((8, _LANES), lambda b, j: (0, 0)),
            pl.BlockSpec((8, _B, _LANES), lambda b, j: (0, 0, 0)),
        ],
        out_shape=[
            jax.ShapeDtypeStruct((_B, _ROWS_PER_B, _LANES), jnp.float32),
            jax.ShapeDtypeStruct((8, _LANES), jnp.float32),
            jax.ShapeDtypeStruct((8, _LANES), jnp.float32),
            jax.ShapeDtypeStruct((8, _B, _LANES), jnp.float32),
        ],
        compiler_params=pltpu.CompilerParams(
            dimension_semantics=("arbitrary", "arbitrary"),
        ),
    )(y_pr4, y_gt4)


@functools.lru_cache(maxsize=1)
def _build_sc_hist():
    @functools.partial(
        pl.kernel,
        out_type=jax.ShapeDtypeStruct((_WRK, _NB), jnp.float32),
        mesh=plsc.VectorSubcoreMesh(core_axis_name="c", subcore_axis_name="s"),
        scratch_types=[
            pltpu.VMEM((_CHUNK,), jnp.float32),
            pltpu.VMEM((_CHUNK,), jnp.float32),
            pltpu.VMEM((_NB * _LN,), jnp.float32),
            pltpu.VMEM((_NB,), jnp.float32),
            pltpu.SemaphoreType.DMA,
            pltpu.SemaphoreType.DMA,
        ],
        compiler_params=pltpu.CompilerParams(needs_layout_passes=False),
    )
    def _sc_hist(neg_hbm, out_hbm, chunk_a, chunk_b, hist_v, merged_v,
                 sem_a, sem_b):
        wid = lax.axis_index("s") * 2 + lax.axis_index("c")
        lane_base = lax.iota(jnp.int32, 16) * _NB
        ones = jnp.ones((16,), jnp.float32)
        zeros16 = jnp.zeros((16,), jnp.float32)

        def zero_body(i, carry):
            for u in range(8):
                hist_v[pl.ds(i * 128 + u * 16, 16)] = zeros16
            return carry

        lax.fori_loop(0, _NB * _LN // 128, zero_body, 0)

        base = wid * _PER_W
        bufs = (chunk_a, chunk_b)
        sems = (sem_a, sem_b)

        def start(ci):
            pltpu.async_copy(neg_hbm.at[pl.ds(base + ci * _CHUNK, _CHUNK)],
                             bufs[ci % 2], sems[ci % 2])

        start(0)
        for ci in range(_NCHUNK):
            pltpu.make_async_copy(
                neg_hbm.at[pl.ds(base + ci * _CHUNK, _CHUNK)],
                bufs[ci % 2], sems[ci % 2]).wait()
            if ci + 1 < _NCHUNK:
                start(ci + 1)
            chunk_v = bufs[ci % 2]

            def body(i, carry):
                for u in range(8):
                    v = chunk_v[pl.ds(i * 128 + u * 16, 16)]
                    bn = jnp.clip((v * _BIN_SCALE).astype(jnp.int32),
                                  0, _NB - 1)
                    plsc.addupdate_scatter(hist_v, [lane_base + bn], ones)
                return carry

            lax.fori_loop(0, _CHUNK // 128, body, 0)

        def merge_body(i, carry):
            acc = zeros16
            for l in range(_LN):
                acc = acc + hist_v[pl.ds(l * _NB + i * 16, 16)]
            merged_v[pl.ds(i * 16, 16)] = acc
            return carry

        lax.fori_loop(0, _NB // 16, merge_body, 0)

        pltpu.sync_copy(merged_v, out_hbm.at[wid])

    return _sc_hist


def _fin_body(neg_ref, hist_ref, sum_ref, mm_ref, dice_ref, out_ref):
    l1_num = jnp.sum(sum_ref[0, :])
    l1_den = jnp.sum(sum_ref[1, :])
    pos_cnt = jnp.sum(sum_ref[2, :])
    neg_cnt_raw = jnp.sum(sum_ref[3, :])
    pos_loss_sum = jnp.sum(sum_ref[4, :])
    dmin = jnp.min(mm_ref[0, :])
    dmax = jnp.max(mm_ref[1, :])

    k_f = jnp.minimum(neg_cnt_raw, pos_cnt * _NEG_RATIO)
    k_i = k_f.astype(jnp.int32)
    k_if = k_i.astype(jnp.float32)

    hist_tot = jnp.sum(hist_ref[...], axis=0)
    bin2 = (lax.broadcasted_iota(jnp.int32, (4, _LANES), 0) * _LANES
            + lax.broadcasted_iota(jnp.int32, (4, _LANES), 1))

    def suffix_count(bidx):
        return jnp.sum(jnp.where(bin2 >= bidx, hist_tot, 0.0))

    def bin_bisect(_, carry):
        lo, hi = carry
        mid = (lo + hi) // 2
        pred = suffix_count(mid) >= k_if
        return (jnp.where(pred, mid, lo), jnp.where(pred, hi, mid))

    b_lo, b_hi = lax.fori_loop(0, 12, bin_bisect,
                               (jnp.int32(0), jnp.int32(_NB)))

    t_lo = b_lo.astype(jnp.float32) * _BIN_W
    t_hi = b_hi.astype(jnp.float32) * _BIN_W

    zed = jnp.zeros((_CH, _LANES), jnp.float32)

    def count_gt(t):
        def chunk(i, acc):
            blk = neg_ref[pl.ds(i * _CH, _CH), :]
            return acc + (blk > t).astype(jnp.float32)
        return jnp.sum(lax.fori_loop(0, _NCH, chunk, zed))

    def refine(_, carry):
        lo, hi = carry
        mid = 0.5 * (lo + hi)
        pred = count_gt(mid) > k_if
        return (jnp.where(pred, mid, lo), jnp.where(pred, hi, mid))

    _, t = lax.fori_loop(0, _REFINE_ITERS, refine, (t_lo, t_hi))

    def final_chunk(i, carry):
        c, s = carry
        blk = neg_ref[pl.ds(i * _CH, _CH), :]
        m = blk > t
        return (c + m.astype(jnp.float32), s + jnp.where(m, blk, 0.0))

    cnt_v, ssum_v = lax.fori_loop(0, _NCH, final_chunk, (zed, zed))
    cnt = jnp.sum(cnt_v)
    ssum = jnp.sum(ssum_v)
    topk_sum = ssum + (k_if - cnt) * t

    balanced = (pos_loss_sum + topk_sum) / (pos_cnt + k_f + _EPS)
    balanced = balanced * _BAL_SCALE

    a = 1.0 / (dmax - dmin)
    c0 = 1.0 - dmin * a
    dice_total = jnp.float32(0.0)
    for bb in range(_B):
        s_ptm = jnp.sum(dice_ref[0, bb, :])
        s_ptml = jnp.sum(dice_ref[1, bb, :])
        s_ppm = jnp.sum(dice_ref[2, bb, :])
        s_ppml = jnp.sum(dice_ref[3, bb, :])
        s_ttm = jnp.sum(dice_ref[4, bb, :])
        s_ttml = jnp.sum(dice_ref[5, bb, :])
        inter = a * s_ptml + c0 * s_ptm
        union = a * (s_ppml + s_ttml) + c0 * (s_ppm + s_ttm) + 2.0 * _DICE_EPS
        dice_total += 1.0 - 2.0 * inter / union
    dice = dice_total / _B

    l1 = jnp.where(l1_den > 0, l1_num / l1_den, jnp.float32(0.0)) * _L1_SCALE

    out_ref[0, 0] = l1 + balanced + dice


def _finalize(neg, hist3, sums, mm, dice):
    return pl.pallas_call(
        _fin_body,
        out_specs=pl.BlockSpec(memory_space=pltpu.SMEM),
        out_shape=jax.ShapeDtypeStruct((1, 1), jnp.float32),
    )(neg, hist3, sums, mm, dice)


def kernel(y_pr, y_gt):
    y_pr4 = y_pr.reshape(_B, 3, _ROWS_PER_B, _LANES)
    y_gt4 = y_gt.reshape(_B, 4, _ROWS_PER_B, _LANES)
    neg, sums, mm, dice = _stream(y_pr4, y_gt4)
    hist = _build_sc_hist()(neg.reshape(_N))
    out = _finalize(neg.reshape(_ROWS, _LANES),
                    hist.reshape(_WRK, 4, _LANES), sums, mm, dice)
    return out[0, 0]


# speedup vs baseline: 1.3250x; 1.2505x over previous
import functools

import jax
import jax.numpy as jnp
from jax import lax
from jax.experimental import pallas as pl
from jax.experimental.pallas import tpu as pltpu
from jax.experimental.pallas import tpu_sc as plsc

_NEG_RATIO = 3.0
_BAL_SCALE = 5.0
_EPS = 1e-09
_L1_SCALE = 10.0
_DICE_EPS = 0.001

_B = 8
_ROWS_PER_B = 400
_LANES = 1024
_RB = 80
_JB = _ROWS_PER_B // _RB
_ROWS = _B * _ROWS_PER_B
_N = _ROWS * _LANES

_CH = 16
_NCH = _ROWS // _CH

_BIG = 3.0e38

_NB = 4096
_VMAX = 27.632
_BIN_SCALE = _NB / _VMAX
_BIN_W = _VMAX / _NB
_LN = 16
_WRK = 32
_PER_W = _N // _WRK
_CHUNK = 4096
_NCHUNK = _PER_W // _CHUNK
_REFINE_ITERS = 4


def _stream_body(pr_ref, gt_ref, neg_ref, sum_ref, mm_ref, dice_ref):
    b = pl.program_id(0)
    j = pl.program_id(1)

    @pl.when(jnp.logical_and(b == 0, j == 0))
    def _init():
        sum_ref[...] = jnp.zeros((8, _LANES), jnp.float32)
        dice_ref[...] = jnp.zeros((8, _B, _LANES), jnp.float32)
        mm_ref[...] = jnp.zeros((8, _LANES), jnp.float32)
        mm_ref[0:1, :] = jnp.full((1, _LANES), _BIG, jnp.float32)
        mm_ref[1:2, :] = jnp.full((1, _LANES), -_BIG, jnp.float32)

    binary = pr_ref[0, 0]
    thresh_binary = pr_ref[0, 1]
    thresh = pr_ref[0, 2]
    gt = gt_ref[0, 0]
    mask = gt_ref[0, 1]
    thresh_map = gt_ref[0, 2]
    thresh_mask = gt_ref[0, 3]

    p = jnp.clip(binary, 1e-12, 1.0 - 1e-12)
    logp = jnp.clip(jnp.log(p), -100.0, None)
    log1mp = jnp.clip(jnp.log(1.0 - p), -100.0, None)
    loss = -(gt * logp + (1.0 - gt) * log1mp)

    pos_m = gt * mask
    neg_m = (1.0 - gt) * mask
    negl = loss * neg_m

    neg_ref[0] = negl

    def _acc_sum(row, val2d):
        sum_ref[row:row + 1, :] += jnp.sum(val2d, axis=0, keepdims=True)

    _acc_sum(0, jnp.abs(thresh - thresh_map) * thresh_mask)
    _acc_sum(1, thresh_mask)
    _acc_sum(2, pos_m)
    _acc_sum(3, neg_m)
    _acc_sum(4, loss * pos_m)

    mm_ref[0:1, :] = jnp.minimum(mm_ref[0:1, :],
                                 jnp.min(loss, axis=0, keepdims=True))
    mm_ref[1:2, :] = jnp.maximum(mm_ref[1:2, :],
                                 jnp.max(loss, axis=0, keepdims=True))

    ptm = thresh_binary * gt * mask
    ppm = thresh_binary * thresh_binary * mask
    ttm = gt * gt * mask

    def _acc_dice(q, val2d):
        dice_ref[q, pl.ds(b, 1), :] += jnp.sum(val2d, axis=0, keepdims=True)

    _acc_dice(0, ptm)
    _acc_dice(1, ptm * loss)
    _acc_dice(2, ppm)
    _acc_dice(3, ppm * loss)
    _acc_dice(4, ttm)
    _acc_dice(5, ttm * loss)


def _stream(y_pr4, y_gt4):
    return pl.pallas_call(
        _stream_body,
        grid=(_B, _JB),
        in_specs=[
            pl.BlockSpec((1, 3, _RB, _LANES), lambda b, j: (b, 0, j, 0)),
            pl.BlockSpec((1, 4, _RB, _LANES), lambda b, j: (b, 0, j, 0)),
        ],
        out_specs=[
            pl.BlockSpec((1, _RB, _LANES), lambda b, j: (b, j, 0)),
            pl.BlockSpec((8, _LANES), lambda b, j: (0, 0)),
            pl.BlockSpec((8, _LANES), lambda b, j: (0, 0)),
            pl.BlockSpec((8, _B, _LANES), lambda b, j: (0, 0, 0)),
        ],
        out_shape=[
            jax.ShapeDtypeStruct((_B, _ROWS_PER_B, _LANES), jnp.float32),
            jax.ShapeDtypeStruct((8, _LANES), jnp.float32),
            jax.ShapeDtypeStruct((8, _LANES), jnp.float32),
            jax.ShapeDtypeStruct((8, _B, _LANES), jnp.float32),
        ],
        compiler_params=pltpu.CompilerParams(
            dimension_semantics=("arbitrary", "arbitrary"),
        ),
    )(y_pr4, y_gt4)


@functools.lru_cache(maxsize=1)
def _build_sc_hist():
    @functools.partial(
        pl.kernel,
        out_type=jax.ShapeDtypeStruct((_WRK, _NB), jnp.float32),
        mesh=plsc.VectorSubcoreMesh(core_axis_name="c", subcore_axis_name="s"),
        scratch_types=[
            pltpu.VMEM((_CHUNK,), jnp.float32),
            pltpu.VMEM((_CHUNK,), jnp.float32),
            pltpu.VMEM((_NB * _LN,), jnp.float32),
            pltpu.VMEM((_NB,), jnp.float32),
            pltpu.SemaphoreType.DMA,
            pltpu.SemaphoreType.DMA,
        ],
        compiler_params=pltpu.CompilerParams(needs_layout_passes=False),
    )
    def _sc_hist(neg_hbm, out_hbm, chunk_a, chunk_b, hist_v, merged_v,
                 sem_a, sem_b):
        wid = lax.axis_index("s") * 2 + lax.axis_index("c")
        lane_base = lax.iota(jnp.int32, 16) * _NB
        ones = jnp.ones((16,), jnp.float32)
        zeros16 = jnp.zeros((16,), jnp.float32)

        def zero_body(i, carry):
            for u in range(8):
                hist_v[pl.ds(i * 128 + u * 16, 16)] = zeros16
            return carry

        lax.fori_loop(0, _NB * _LN // 128, zero_body, 0)

        base = wid * _PER_W
        bufs = (chunk_a, chunk_b)
        sems = (sem_a, sem_b)

        def start(ci):
            pltpu.async_copy(neg_hbm.at[pl.ds(base + ci * _CHUNK, _CHUNK)],
                             bufs[ci % 2], sems[ci % 2])

        start(0)
        for ci in range(_NCHUNK):
            pltpu.make_async_copy(
                neg_hbm.at[pl.ds(base + ci * _CHUNK, _CHUNK)],
                bufs[ci % 2], sems[ci % 2]).wait()
            if ci + 1 < _NCHUNK:
                start(ci + 1)
            chunk_v = bufs[ci % 2]

            @plsc.parallel_loop(0, _CHUNK // 16, unroll=8)
            def _hist_body(i, chunk_v=chunk_v):
                v = chunk_v[pl.ds(i * 16, 16)]
                bn = jnp.clip((v * _BIN_SCALE).astype(jnp.int32),
                              0, _NB - 1)
                plsc.addupdate_scatter(hist_v, [lane_base + bn], ones)

        def merge_body(i, carry):
            acc = zeros16
            for l in range(_LN):
                acc = acc + hist_v[pl.ds(l * _NB + i * 16, 16)]
            merged_v[pl.ds(i * 16, 16)] = acc
            return carry

        lax.fori_loop(0, _NB // 16, merge_body, 0)

        pltpu.sync_copy(merged_v, out_hbm.at[wid])

    return _sc_hist


def _fin_body(neg_ref, hist_ref, sum_ref, mm_ref, dice_ref, out_ref):
    l1_num = jnp.sum(sum_ref[0, :])
    l1_den = jnp.sum(sum_ref[1, :])
    pos_cnt = jnp.sum(sum_ref[2, :])
    neg_cnt_raw = jnp.sum(sum_ref[3, :])
    pos_loss_sum = jnp.sum(sum_ref[4, :])
    dmin = jnp.min(mm_ref[0, :])
    dmax = jnp.max(mm_ref[1, :])

    k_f = jnp.minimum(neg_cnt_raw, pos_cnt * _NEG_RATIO)
    k_i = k_f.astype(jnp.int32)
    k_if = k_i.astype(jnp.float32)

    hist_tot = jnp.sum(hist_ref[...], axis=0)
    bin2 = (lax.broadcasted_iota(jnp.int32, (4, _LANES), 0) * _LANES
            + lax.broadcasted_iota(jnp.int32, (4, _LANES), 1))

    def suffix_count(bidx):
        return jnp.sum(jnp.where(bin2 >= bidx, hist_tot, 0.0))

    def bin_bisect(_, carry):
        lo, hi = carry
        mid = (lo + hi) // 2
        pred = suffix_count(mid) >= k_if
        return (jnp.where(pred, mid, lo), jnp.where(pred, hi, mid))

    b_lo, b_hi = lax.fori_loop(0, 12, bin_bisect,
                               (jnp.int32(0), jnp.int32(_NB)))

    t_lo = b_lo.astype(jnp.float32) * _BIN_W
    t_hi = b_hi.astype(jnp.float32) * _BIN_W

    zed = jnp.zeros((_CH, _LANES), jnp.float32)

    def count_gt(t):
        def chunk(i, acc):
            blk = neg_ref[pl.ds(i * _CH, _CH), :]
            return acc + (blk > t).astype(jnp.float32)
        return jnp.sum(lax.fori_loop(0, _NCH, chunk, zed))

    def refine(_, carry):
        lo, hi = carry
        mid = 0.5 * (lo + hi)
        pred = count_gt(mid) > k_if
        return (jnp.where(pred, mid, lo), jnp.where(pred, hi, mid))

    _, t = lax.fori_loop(0, _REFINE_ITERS, refine, (t_lo, t_hi))

    def final_chunk(i, carry):
        c, s = carry
        blk = neg_ref[pl.ds(i * _CH, _CH), :]
        m = blk > t
        return (c + m.astype(jnp.float32), s + jnp.where(m, blk, 0.0))

    cnt_v, ssum_v = lax.fori_loop(0, _NCH, final_chunk, (zed, zed))
    cnt = jnp.sum(cnt_v)
    ssum = jnp.sum(ssum_v)
    topk_sum = ssum + (k_if - cnt) * t

    balanced = (pos_loss_sum + topk_sum) / (pos_cnt + k_f + _EPS)
    balanced = balanced * _BAL_SCALE

    a = 1.0 / (dmax - dmin)
    c0 = 1.0 - dmin * a
    dice_total = jnp.float32(0.0)
    for bb in range(_B):
        s_ptm = jnp.sum(dice_ref[0, bb, :])
        s_ptml = jnp.sum(dice_ref[1, bb, :])
        s_ppm = jnp.sum(dice_ref[2, bb, :])
        s_ppml = jnp.sum(dice_ref[3, bb, :])
        s_ttm = jnp.sum(dice_ref[4, bb, :])
        s_ttml = jnp.sum(dice_ref[5, bb, :])
        inter = a * s_ptml + c0 * s_ptm
        union = a * (s_ppml + s_ttml) + c0 * (s_ppm + s_ttm) + 2.0 * _DICE_EPS
        dice_total += 1.0 - 2.0 * inter / union
    dice = dice_total / _B

    l1 = jnp.where(l1_den > 0, l1_num / l1_den, jnp.float32(0.0)) * _L1_SCALE

    out_ref[0, 0] = l1 + balanced + dice


def _finalize(neg, hist3, sums, mm, dice):
    return pl.pallas_call(
        _fin_body,
        out_specs=pl.BlockSpec(memory_space=pltpu.SMEM),
        out_shape=jax.ShapeDtypeStruct((1, 1), jnp.float32),
    )(neg, hist3, sums, mm, dice)


def kernel(y_pr, y_gt):
    y_pr4 = y_pr.reshape(_B, 3, _ROWS_PER_B, _LANES)
    y_gt4 = y_gt.reshape(_B, 4, _ROWS_PER_B, _LANES)
    neg, sums, mm, dice = _stream(y_pr4, y_gt4)
    hist = _build_sc_hist()(neg.reshape(_N))
    out = _finalize(neg.reshape(_ROWS, _LANES),
                    hist.reshape(_WRK, 4, _LANES), sums, mm, dice)
    return out[0, 0]


# speedup vs baseline: 1.4113x; 1.0652x over previous
import functools

import jax
import jax.numpy as jnp
from jax import lax
from jax.experimental import pallas as pl
from jax.experimental.pallas import tpu as pltpu
from jax.experimental.pallas import tpu_sc as plsc

_NEG_RATIO = 3.0
_BAL_SCALE = 5.0
_EPS = 1e-09
_L1_SCALE = 10.0
_DICE_EPS = 0.001

_B = 8
_ROWS_PER_B = 400
_LANES = 1024
_RB = 80
_JB = _ROWS_PER_B // _RB
_ROWS = _B * _ROWS_PER_B
_N = _ROWS * _LANES

_BIG = 3.0e38

_NB = 2048
_VMAX = 27.632
_BIN_SCALE = _NB / _VMAX
_BIN_W = _VMAX / _NB
_LN = 16
_WRK = 32
_PER_W = _N // _WRK
_CHUNK = 4096
_NCHUNK = _PER_W // _CHUNK
_NBROW = _NB // _LANES


def _stream_body(pr_ref, gt_ref, neg_ref, sum_ref, mm_ref, dice_ref):
    b = pl.program_id(0)
    j = pl.program_id(1)

    @pl.when(jnp.logical_and(b == 0, j == 0))
    def _init():
        sum_ref[...] = jnp.zeros((8, _LANES), jnp.float32)
        dice_ref[...] = jnp.zeros((8, _B, _LANES), jnp.float32)
        mm_ref[...] = jnp.zeros((8, _LANES), jnp.float32)
        mm_ref[0:1, :] = jnp.full((1, _LANES), _BIG, jnp.float32)
        mm_ref[1:2, :] = jnp.full((1, _LANES), -_BIG, jnp.float32)

    binary = pr_ref[0, 0]
    thresh_binary = pr_ref[0, 1]
    thresh = pr_ref[0, 2]
    gt = gt_ref[0, 0]
    mask = gt_ref[0, 1]
    thresh_map = gt_ref[0, 2]
    thresh_mask = gt_ref[0, 3]

    p = jnp.clip(binary, 1e-12, 1.0 - 1e-12)
    logp = jnp.clip(jnp.log(p), -100.0, None)
    log1mp = jnp.clip(jnp.log(1.0 - p), -100.0, None)
    loss = -(gt * logp + (1.0 - gt) * log1mp)

    pos_m = gt * mask
    neg_m = (1.0 - gt) * mask
    negl = loss * neg_m

    neg_ref[0] = negl

    def _acc_sum(row, val2d):
        sum_ref[row:row + 1, :] += jnp.sum(val2d, axis=0, keepdims=True)

    _acc_sum(0, jnp.abs(thresh - thresh_map) * thresh_mask)
    _acc_sum(1, thresh_mask)
    _acc_sum(2, pos_m)
    _acc_sum(3, neg_m)
    _acc_sum(4, loss * pos_m)

    mm_ref[0:1, :] = jnp.minimum(mm_ref[0:1, :],
                                 jnp.min(loss, axis=0, keepdims=True))
    mm_ref[1:2, :] = jnp.maximum(mm_ref[1:2, :],
                                 jnp.max(loss, axis=0, keepdims=True))

    ptm = thresh_binary * gt * mask
    ppm = thresh_binary * thresh_binary * mask
    ttm = gt * gt * mask

    def _acc_dice(q, val2d):
        dice_ref[q, pl.ds(b, 1), :] += jnp.sum(val2d, axis=0, keepdims=True)

    _acc_dice(0, ptm)
    _acc_dice(1, ptm * loss)
    _acc_dice(2, ppm)
    _acc_dice(3, ppm * loss)
    _acc_dice(4, ttm)
    _acc_dice(5, ttm * loss)


def _stream(y_pr4, y_gt4):
    return pl.pallas_call(
        _stream_body,
        grid=(_B, _JB),
        in_specs=[
            pl.BlockSpec((1, 3, _RB, _LANES), lambda b, j: (b, 0, j, 0)),
            pl.BlockSpec((1, 4, _RB, _LANES), lambda b, j: (b, 0, j, 0)),
        ],
        out_specs=[
            pl.BlockSpec((1, _RB, _LANES), lambda b, j: (b, j, 0)),
            pl.BlockSpec((8, _LANES), lambda b, j: (0, 0)),
            pl.BlockSpec((8, _LANES), lambda b, j: (0, 0)),
            pl.BlockSpec((8, _B, _LANES), lambda b, j: (0, 0, 0)),
        ],
        out_shape=[
            jax.ShapeDtypeStruct((_B, _ROWS_PER_B, _LANES), jnp.float32),
            jax.ShapeDtypeStruct((8, _LANES), jnp.float32),
            jax.ShapeDtypeStruct((8, _LANES), jnp.float32),
            jax.ShapeDtypeStruct((8, _B, _LANES), jnp.float32),
        ],
        compiler_params=pltpu.CompilerParams(
            dimension_semantics=("arbitrary", "arbitrary"),
        ),
    )(y_pr4, y_gt4)


@functools.lru_cache(maxsize=1)
def _build_sc_hist():
    @functools.partial(
        pl.kernel,
        out_type=(
            jax.ShapeDtypeStruct((_WRK, _NB), jnp.float32),
            jax.ShapeDtypeStruct((_WRK, _NB), jnp.float32),
        ),
        mesh=plsc.VectorSubcoreMesh(core_axis_name="c", subcore_axis_name="s"),
        scratch_types=[
            pltpu.VMEM((_CHUNK,), jnp.float32),
            pltpu.VMEM((_CHUNK,), jnp.float32),
            pltpu.VMEM((_NB * _LN,), jnp.float32),
            pltpu.VMEM((_NB * _LN,), jnp.float32),
            pltpu.VMEM((_NB,), jnp.float32),
            pltpu.VMEM((_NB,), jnp.float32),
            pltpu.SemaphoreType.DMA,
            pltpu.SemaphoreType.DMA,
        ],
        compiler_params=pltpu.CompilerParams(needs_layout_passes=False),
    )
    def _sc_hist(neg_hbm, cnt_hbm, sum_hbm, chunk_a, chunk_b,
                 hcnt_v, hsum_v, mcnt_v, msum_v, sem_a, sem_b):
        wid = lax.axis_index("s") * 2 + lax.axis_index("c")
        lane_base = lax.iota(jnp.int32, 16) * _NB
        ones = jnp.ones((16,), jnp.float32)
        zeros16 = jnp.zeros((16,), jnp.float32)

        def zero_body(i, carry):
            for u in range(8):
                hcnt_v[pl.ds(i * 128 + u * 16, 16)] = zeros16
                hsum_v[pl.ds(i * 128 + u * 16, 16)] = zeros16
            return carry

        lax.fori_loop(0, _NB * _LN // 128, zero_body, 0)

        base = wid * _PER_W
        bufs = (chunk_a, chunk_b)
        sems = (sem_a, sem_b)

        def start(ci):
            pltpu.async_copy(neg_hbm.at[pl.ds(base + ci * _CHUNK, _CHUNK)],
                             bufs[ci % 2], sems[ci % 2])

        start(0)
        for ci in range(_NCHUNK):
            pltpu.make_async_copy(
                neg_hbm.at[pl.ds(base + ci * _CHUNK, _CHUNK)],
                bufs[ci % 2], sems[ci % 2]).wait()
            if ci + 1 < _NCHUNK:
                start(ci + 1)
            chunk_v = bufs[ci % 2]

            @plsc.parallel_loop(0, _CHUNK // 16, unroll=8)
            def _hist_body(i, chunk_v=chunk_v):
                v = chunk_v[pl.ds(i * 16, 16)]
                bn = jnp.clip((v * _BIN_SCALE).astype(jnp.int32),
                              0, _NB - 1)
                idx = lane_base + bn
                plsc.addupdate_scatter(hcnt_v, [idx], ones)
                plsc.addupdate_scatter(hsum_v, [idx], v)

        def merge_body(i, carry):
            acc_c = zeros16
            acc_s = zeros16
            for l in range(_LN):
                acc_c = acc_c + hcnt_v[pl.ds(l * _NB + i * 16, 16)]
                acc_s = acc_s + hsum_v[pl.ds(l * _NB + i * 16, 16)]
            mcnt_v[pl.ds(i * 16, 16)] = acc_c
            msum_v[pl.ds(i * 16, 16)] = acc_s
            return carry

        lax.fori_loop(0, _NB // 16, merge_body, 0)

        pltpu.sync_copy(mcnt_v, cnt_hbm.at[wid])
        pltpu.sync_copy(msum_v, sum_hbm.at[wid])

    return _sc_hist


def _fin_body(hcnt_ref, hsum_ref, sum_ref, mm_ref, dice_ref, out_ref):
    l1_num = jnp.sum(sum_ref[0, :])
    l1_den = jnp.sum(sum_ref[1, :])
    pos_cnt = jnp.sum(sum_ref[2, :])
    neg_cnt_raw = jnp.sum(sum_ref[3, :])
    pos_loss_sum = jnp.sum(sum_ref[4, :])
    dmin = jnp.min(mm_ref[0, :])
    dmax = jnp.max(mm_ref[1, :])

    k_f = jnp.minimum(neg_cnt_raw, pos_cnt * _NEG_RATIO)
    k_i = k_f.astype(jnp.int32)
    k_if = k_i.astype(jnp.float32)

    cnt_tot = jnp.sum(hcnt_ref[...], axis=0)
    sum_tot = jnp.sum(hsum_ref[...], axis=0)
    bin2 = (lax.broadcasted_iota(jnp.int32, (_NBROW, _LANES), 0) * _LANES
            + lax.broadcasted_iota(jnp.int32, (_NBROW, _LANES), 1))

    def suffix(bidx):
        sel = bin2 >= bidx
        c = jnp.sum(jnp.where(sel, cnt_tot, 0.0))
        s = jnp.sum(jnp.where(sel, sum_tot, 0.0))
        return c, s

    def bin_bisect(_, carry):
        lo, hi = carry
        mid = (lo + hi) // 2
        c, _s = suffix(mid)
        pred = c >= k_if
        return (jnp.where(pred, mid, lo), jnp.where(pred, hi, mid))

    b_lo, _ = lax.fori_loop(0, 11, bin_bisect,
                            (jnp.int32(0), jnp.int32(_NB)))

    c_at, s_at = suffix(b_lo)
    c_above, s_above = suffix(b_lo + 1)
    c_in = c_at - c_above
    s_in = s_at - s_above
    take = k_if - c_above
    mean_in = s_in / jnp.maximum(c_in, 1.0)
    topk_sum = s_above + take * mean_in

    balanced = (pos_loss_sum + topk_sum) / (pos_cnt + k_f + _EPS)
    balanced = balanced * _BAL_SCALE

    a = 1.0 / (dmax - dmin)
    c0 = 1.0 - dmin * a
    dice_total = jnp.float32(0.0)
    for bb in range(_B):
        s_ptm = jnp.sum(dice_ref[0, bb, :])
        s_ptml = jnp.sum(dice_ref[1, bb, :])
        s_ppm = jnp.sum(dice_ref[2, bb, :])
        s_ppml = jnp.sum(dice_ref[3, bb, :])
        s_ttm = jnp.sum(dice_ref[4, bb, :])
        s_ttml = jnp.sum(dice_ref[5, bb, :])
        inter = a * s_ptml + c0 * s_ptm
        union = a * (s_ppml + s_ttml) + c0 * (s_ppm + s_ttm) + 2.0 * _DICE_EPS
        dice_total += 1.0 - 2.0 * inter / union
    dice = dice_total / _B

    l1 = jnp.where(l1_den > 0, l1_num / l1_den, jnp.float32(0.0)) * _L1_SCALE

    out_ref[0, 0] = l1 + balanced + dice


def _finalize(hcnt3, hsum3, sums, mm, dice):
    return pl.pallas_call(
        _fin_body,
        out_specs=pl.BlockSpec(memory_space=pltpu.SMEM),
        out_shape=jax.ShapeDtypeStruct((1, 1), jnp.float32),
    )(hcnt3, hsum3, sums, mm, dice)


def kernel(y_pr, y_gt):
    y_pr4 = y_pr.reshape(_B, 3, _ROWS_PER_B, _LANES)
    y_gt4 = y_gt.reshape(_B, 4, _ROWS_PER_B, _LANES)
    neg, sums, mm, dice = _stream(y_pr4, y_gt4)
    hcnt, hsum = _build_sc_hist()(neg.reshape(_N))
    out = _finalize(hcnt.reshape(_WRK, _NBROW, _LANES),
                    hsum.reshape(_WRK, _NBROW, _LANES), sums, mm, dice)
    return out[0, 0]


# speedup vs baseline: 1.5465x; 1.0958x over previous
import functools

import jax
import jax.numpy as jnp
from jax import lax
from jax.experimental import pallas as pl
from jax.experimental.pallas import tpu as pltpu
from jax.experimental.pallas import tpu_sc as plsc

_NEG_RATIO = 3.0
_BAL_SCALE = 5.0
_EPS = 1e-09
_L1_SCALE = 10.0
_DICE_EPS = 0.001

_B = 8
_ROWS_PER_B = 400
_LANES = 1024
_RB = 80
_JB = _ROWS_PER_B // _RB
_ROWS = _B * _ROWS_PER_B
_N = _ROWS * _LANES

_BIG = 3.0e38

_NB = 2048
_VMAX = 27.632
_BIN_SCALE = _NB / _VMAX
_BIN_W = _VMAX / _NB
_LN = 16
_WRK = 32
_NBROW = _NB // _LANES

_GROUP_ROWS = 400
_BAND = 256
_UROWS = 40
_UNITS = _GROUP_ROWS // _UROWS
_UVEC = _UROWS * _BAND // 16


def _stream_body(pr_ref, gt_ref, neg_ref, sum_ref, mm_ref, dice_ref):
    b = pl.program_id(0)
    j = pl.program_id(1)

    @pl.when(jnp.logical_and(b == 0, j == 0))
    def _init():
        sum_ref[...] = jnp.zeros((8, _LANES), jnp.float32)
        dice_ref[...] = jnp.zeros((8, _B, _LANES), jnp.float32)
        mm_ref[...] = jnp.zeros((8, _LANES), jnp.float32)
        mm_ref[0:1, :] = jnp.full((1, _LANES), _BIG, jnp.float32)
        mm_ref[1:2, :] = jnp.full((1, _LANES), -_BIG, jnp.float32)

    binary = pr_ref[0, 0]
    thresh_binary = pr_ref[0, 1]
    thresh = pr_ref[0, 2]
    gt = gt_ref[0, 0]
    mask = gt_ref[0, 1]
    thresh_map = gt_ref[0, 2]
    thresh_mask = gt_ref[0, 3]

    p = jnp.clip(binary, 1e-12, 1.0 - 1e-12)
    logp = jnp.clip(jnp.log(p), -100.0, None)
    log1mp = jnp.clip(jnp.log(1.0 - p), -100.0, None)
    loss = -(gt * logp + (1.0 - gt) * log1mp)

    pos_m = gt * mask
    neg_m = (1.0 - gt) * mask
    negl = loss * neg_m

    neg_ref[0] = negl

    def _acc_sum(row, val2d):
        sum_ref[row:row + 1, :] += jnp.sum(val2d, axis=0, keepdims=True)

    _acc_sum(0, jnp.abs(thresh - thresh_map) * thresh_mask)
    _acc_sum(1, thresh_mask)
    _acc_sum(2, pos_m)
    _acc_sum(3, neg_m)
    _acc_sum(4, loss * pos_m)

    mm_ref[0:1, :] = jnp.minimum(mm_ref[0:1, :],
                                 jnp.min(loss, axis=0, keepdims=True))
    mm_ref[1:2, :] = jnp.maximum(mm_ref[1:2, :],
                                 jnp.max(loss, axis=0, keepdims=True))

    ptm = thresh_binary * gt * mask
    ppm = thresh_binary * thresh_binary * mask
    ttm = gt * gt * mask

    def _acc_dice(q, val2d):
        dice_ref[q, pl.ds(b, 1), :] += jnp.sum(val2d, axis=0, keepdims=True)

    _acc_dice(0, ptm)
    _acc_dice(1, ptm * loss)
    _acc_dice(2, ppm)
    _acc_dice(3, ppm * loss)
    _acc_dice(4, ttm)
    _acc_dice(5, ttm * loss)


def _stream(y_pr4, y_gt4):
    return pl.pallas_call(
        _stream_body,
        grid=(_B, _JB),
        in_specs=[
            pl.BlockSpec((1, 3, _RB, _LANES), lambda b, j: (b, 0, j, 0)),
            pl.BlockSpec((1, 4, _RB, _LANES), lambda b, j: (b, 0, j, 0)),
        ],
        out_specs=[
            pl.BlockSpec((1, _RB, _LANES), lambda b, j: (b, j, 0)),
            pl.BlockSpec((8, _LANES), lambda b, j: (0, 0)),
            pl.BlockSpec((8, _LANES), lambda b, j: (0, 0)),
            pl.BlockSpec((8, _B, _LANES), lambda b, j: (0, 0, 0)),
        ],
        out_shape=[
            jax.ShapeDtypeStruct((_B, _ROWS_PER_B, _LANES), jnp.float32),
            jax.ShapeDtypeStruct((8, _LANES), jnp.float32),
            jax.ShapeDtypeStruct((8, _LANES), jnp.float32),
            jax.ShapeDtypeStruct((8, _B, _LANES), jnp.float32),
        ],
        compiler_params=pltpu.CompilerParams(
            dimension_semantics=("arbitrary", "arbitrary"),
        ),
    )(y_pr4, y_gt4)


@functools.lru_cache(maxsize=1)
def _build_sc_hist():
    @functools.partial(
        pl.kernel,
        out_type=(
            jax.ShapeDtypeStruct((_WRK, _NB), jnp.float32),
            jax.ShapeDtypeStruct((_WRK, _NB), jnp.float32),
        ),
        mesh=plsc.VectorSubcoreMesh(core_axis_name="c", subcore_axis_name="s"),
        scratch_types=[
            pltpu.VMEM((_UROWS, _BAND), jnp.float32),
            pltpu.VMEM((_UROWS, _BAND), jnp.float32),
            pltpu.VMEM((_NB * _LN,), jnp.float32),
            pltpu.VMEM((_NB * _LN,), jnp.float32),
            pltpu.VMEM((_NB,), jnp.float32),
            pltpu.VMEM((_NB,), jnp.float32),
            pltpu.SemaphoreType.DMA,
            pltpu.SemaphoreType.DMA,
        ],
        compiler_params=pltpu.CompilerParams(needs_layout_passes=False),
    )
    def _sc_hist(neg_hbm, cnt_hbm, sum_hbm, chunk_a, chunk_b,
                 hcnt_v, hsum_v, mcnt_v, msum_v, sem_a, sem_b):
        wid = lax.axis_index("s") * 2 + lax.axis_index("c")
        lane_base = lax.iota(jnp.int32, 16) * _NB
        ones = jnp.ones((16,), jnp.float32)
        zeros16 = jnp.zeros((16,), jnp.float32)

        def zero_body(i, carry):
            for u in range(8):
                hcnt_v[pl.ds(i * 128 + u * 16, 16)] = zeros16
                hsum_v[pl.ds(i * 128 + u * 16, 16)] = zeros16
            return carry

        lax.fori_loop(0, _NB * _LN // 128, zero_body, 0)

        row0 = (wid >> 2) * _GROUP_ROWS
        col0 = (wid & 3) * _BAND
        bufs = (chunk_a, chunk_b)
        sems = (sem_a, sem_b)

        def start(ci):
            pltpu.async_copy(
                neg_hbm.at[pl.ds(row0 + ci * _UROWS, _UROWS),
                           pl.ds(col0, _BAND)],
                bufs[ci % 2], sems[ci % 2])

        start(0)
        for ci in range(_UNITS):
            pltpu.make_async_copy(
                neg_hbm.at[pl.ds(row0 + ci * _UROWS, _UROWS),
                           pl.ds(col0, _BAND)],
                bufs[ci % 2], sems[ci % 2]).wait()
            if ci + 1 < _UNITS:
                start(ci + 1)
            chunk_v = bufs[ci % 2]

            @plsc.parallel_loop(0, _UVEC, unroll=8)
            def _hist_body(i, chunk_v=chunk_v):
                r = i >> 4
                co = (i & 15) << 4
                v = chunk_v[r, pl.ds(co, 16)]
                bn = jnp.clip((v * _BIN_SCALE).astype(jnp.int32),
                              0, _NB - 1)
                idx = lane_base + bn
                plsc.addupdate_scatter(hcnt_v, [idx], ones)
                plsc.addupdate_scatter(hsum_v, [idx], v)

        def merge_body(i, carry):
            acc_c = zeros16
            acc_s = zeros16
            for l in range(_LN):
                acc_c = acc_c + hcnt_v[pl.ds(l * _NB + i * 16, 16)]
                acc_s = acc_s + hsum_v[pl.ds(l * _NB + i * 16, 16)]
            mcnt_v[pl.ds(i * 16, 16)] = acc_c
            msum_v[pl.ds(i * 16, 16)] = acc_s
            return carry

        lax.fori_loop(0, _NB // 16, merge_body, 0)

        pltpu.sync_copy(mcnt_v, cnt_hbm.at[wid])
        pltpu.sync_copy(msum_v, sum_hbm.at[wid])

    return _sc_hist


def _fin_body(hcnt_ref, hsum_ref, sum_ref, mm_ref, dice_ref, out_ref):
    l1_num = jnp.sum(sum_ref[0, :])
    l1_den = jnp.sum(sum_ref[1, :])
    pos_cnt = jnp.sum(sum_ref[2, :])
    neg_cnt_raw = jnp.sum(sum_ref[3, :])
    pos_loss_sum = jnp.sum(sum_ref[4, :])
    dmin = jnp.min(mm_ref[0, :])
    dmax = jnp.max(mm_ref[1, :])

    k_f = jnp.minimum(neg_cnt_raw, pos_cnt * _NEG_RATIO)
    k_i = k_f.astype(jnp.int32)
    k_if = k_i.astype(jnp.float32)

    cnt_tot = jnp.sum(hcnt_ref[...], axis=0)
    sum_tot = jnp.sum(hsum_ref[...], axis=0)
    bin2 = (lax.broadcasted_iota(jnp.int32, (_NBROW, _LANES), 0) * _LANES
            + lax.broadcasted_iota(jnp.int32, (_NBROW, _LANES), 1))

    def suffix(bidx):
        sel = bin2 >= bidx
        c = jnp.sum(jnp.where(sel, cnt_tot, 0.0))
        s = jnp.sum(jnp.where(sel, sum_tot, 0.0))
        return c, s

    def bin_bisect(_, carry):
        lo, hi = carry
        mid = (lo + hi) // 2
        c, _s = suffix(mid)
        pred = c >= k_if
        return (jnp.where(pred, mid, lo), jnp.where(pred, hi, mid))

    b_lo, _ = lax.fori_loop(0, 11, bin_bisect,
                            (jnp.int32(0), jnp.int32(_NB)))

    c_at, s_at = suffix(b_lo)
    c_above, s_above = suffix(b_lo + 1)
    c_in = c_at - c_above
    s_in = s_at - s_above
    take = k_if - c_above
    mean_in = s_in / jnp.maximum(c_in, 1.0)
    topk_sum = s_above + take * mean_in

    balanced = (pos_loss_sum + topk_sum) / (pos_cnt + k_f + _EPS)
    balanced = balanced * _BAL_SCALE

    a = 1.0 / (dmax - dmin)
    c0 = 1.0 - dmin * a
    dice_total = jnp.float32(0.0)
    for bb in range(_B):
        s_ptm = jnp.sum(dice_ref[0, bb, :])
        s_ptml = jnp.sum(dice_ref[1, bb, :])
        s_ppm = jnp.sum(dice_ref[2, bb, :])
        s_ppml = jnp.sum(dice_ref[3, bb, :])
        s_ttm = jnp.sum(dice_ref[4, bb, :])
        s_ttml = jnp.sum(dice_ref[5, bb, :])
        inter = a * s_ptml + c0 * s_ptm
        union = a * (s_ppml + s_ttml) + c0 * (s_ppm + s_ttm) + 2.0 * _DICE_EPS
        dice_total += 1.0 - 2.0 * inter / union
    dice = dice_total / _B

    l1 = jnp.where(l1_den > 0, l1_num / l1_den, jnp.float32(0.0)) * _L1_SCALE

    out_ref[0, 0] = l1 + balanced + dice


def _finalize(hcnt3, hsum3, sums, mm, dice):
    return pl.pallas_call(
        _fin_body,
        out_specs=pl.BlockSpec(memory_space=pltpu.SMEM),
        out_shape=jax.ShapeDtypeStruct((1, 1), jnp.float32),
    )(hcnt3, hsum3, sums, mm, dice)


def kernel(y_pr, y_gt):
    y_pr4 = y_pr.reshape(_B, 3, _ROWS_PER_B, _LANES)
    y_gt4 = y_gt.reshape(_B, 4, _ROWS_PER_B, _LANES)
    neg, sums, mm, dice = _stream(y_pr4, y_gt4)
    hcnt, hsum = _build_sc_hist()(neg.reshape(_ROWS, _LANES))
    out = _finalize(hcnt.reshape(_WRK, _NBROW, _LANES),
                    hsum.reshape(_WRK, _NBROW, _LANES), sums, mm, dice)
    return out[0, 0]


# speedup vs baseline: 1.6300x; 1.0540x over previous
import functools

import jax
import jax.numpy as jnp
from jax import lax
from jax.experimental import pallas as pl
from jax.experimental.pallas import tpu as pltpu
from jax.experimental.pallas import tpu_sc as plsc

_NEG_RATIO = 3.0
_BAL_SCALE = 5.0
_EPS = 1e-09
_L1_SCALE = 10.0
_DICE_EPS = 0.001

_B = 8
_ROWS_PER_B = 400
_LANES = 1024
_RB = 200
_JB = _ROWS_PER_B // _RB
_ROWS = _B * _ROWS_PER_B
_N = _ROWS * _LANES

_BIG = 3.0e38

_NB = 2048
_VMAX = 27.632
_BIN_SCALE = _NB / _VMAX
_BIN_W = _VMAX / _NB
_LN = 16
_WRK = 32
_NBROW = _NB // _LANES

_GROUP_ROWS = 400
_BAND = 256
_UROWS = 40
_UNITS = _GROUP_ROWS // _UROWS
_UVEC = _UROWS * _BAND // 16


def _stream_body(pr_ref, gt_ref, neg_ref, sum_ref, mm_ref, dice_ref):
    b = pl.program_id(0)
    j = pl.program_id(1)

    @pl.when(jnp.logical_and(b == 0, j == 0))
    def _init():
        sum_ref[...] = jnp.zeros((8, _LANES), jnp.float32)
        dice_ref[...] = jnp.zeros((8, _B, _LANES), jnp.float32)
        mm_ref[...] = jnp.zeros((8, _LANES), jnp.float32)
        mm_ref[0:1, :] = jnp.full((1, _LANES), _BIG, jnp.float32)
        mm_ref[1:2, :] = jnp.full((1, _LANES), -_BIG, jnp.float32)

    binary = pr_ref[0, 0]
    thresh_binary = pr_ref[0, 1]
    thresh = pr_ref[0, 2]
    gt = gt_ref[0, 0]
    mask = gt_ref[0, 1]
    thresh_map = gt_ref[0, 2]
    thresh_mask = gt_ref[0, 3]

    p = jnp.clip(binary, 1e-12, 1.0 - 1e-12)
    logp = jnp.clip(jnp.log(p), -100.0, None)
    log1mp = jnp.clip(jnp.log(1.0 - p), -100.0, None)
    loss = -(gt * logp + (1.0 - gt) * log1mp)

    pos_m = gt * mask
    neg_m = (1.0 - gt) * mask
    negl = loss * neg_m

    neg_ref[0] = negl

    def _acc_sum(row, val2d):
        sum_ref[row:row + 1, :] += jnp.sum(val2d, axis=0, keepdims=True)

    _acc_sum(0, jnp.abs(thresh - thresh_map) * thresh_mask)
    _acc_sum(1, thresh_mask)
    _acc_sum(2, pos_m)
    _acc_sum(3, neg_m)
    _acc_sum(4, loss * pos_m)

    mm_ref[0:1, :] = jnp.minimum(mm_ref[0:1, :],
                                 jnp.min(loss, axis=0, keepdims=True))
    mm_ref[1:2, :] = jnp.maximum(mm_ref[1:2, :],
                                 jnp.max(loss, axis=0, keepdims=True))

    ptm = thresh_binary * gt * mask
    ppm = thresh_binary * thresh_binary * mask
    ttm = gt * gt * mask

    def _acc_dice(q, val2d):
        dice_ref[q, pl.ds(b, 1), :] += jnp.sum(val2d, axis=0, keepdims=True)

    _acc_dice(0, ptm)
    _acc_dice(1, ptm * loss)
    _acc_dice(2, ppm)
    _acc_dice(3, ppm * loss)
    _acc_dice(4, ttm)
    _acc_dice(5, ttm * loss)


def _stream(y_pr4, y_gt4):
    return pl.pallas_call(
        _stream_body,
        grid=(_B, _JB),
        in_specs=[
            pl.BlockSpec((1, 3, _RB, _LANES), lambda b, j: (b, 0, j, 0)),
            pl.BlockSpec((1, 4, _RB, _LANES), lambda b, j: (b, 0, j, 0)),
        ],
        out_specs=[
            pl.BlockSpec((1, _RB, _LANES), lambda b, j: (b, j, 0)),
            pl.BlockSpec((8, _LANES), lambda b, j: (0, 0)),
            pl.BlockSpec((8, _LANES), lambda b, j: (0, 0)),
            pl.BlockSpec((8, _B, _LANES), lambda b, j: (0, 0, 0)),
        ],
        out_shape=[
            jax.ShapeDtypeStruct((_B, _ROWS_PER_B, _LANES), jnp.float32),
            jax.ShapeDtypeStruct((8, _LANES), jnp.float32),
            jax.ShapeDtypeStruct((8, _LANES), jnp.float32),
            jax.ShapeDtypeStruct((8, _B, _LANES), jnp.float32),
        ],
        compiler_params=pltpu.CompilerParams(
            dimension_semantics=("arbitrary", "arbitrary"),
        ),
    )(y_pr4, y_gt4)


@functools.lru_cache(maxsize=1)
def _build_sc_hist():
    @functools.partial(
        pl.kernel,
        out_type=(
            jax.ShapeDtypeStruct((_WRK, _NB), jnp.float32),
            jax.ShapeDtypeStruct((_WRK, _NB), jnp.float32),
        ),
        mesh=plsc.VectorSubcoreMesh(core_axis_name="c", subcore_axis_name="s"),
        scratch_types=[
            pltpu.VMEM((_UROWS, _BAND), jnp.float32),
            pltpu.VMEM((_UROWS, _BAND), jnp.float32),
            pltpu.VMEM((_NB * _LN,), jnp.float32),
            pltpu.VMEM((_NB * _LN,), jnp.float32),
            pltpu.VMEM((_NB,), jnp.float32),
            pltpu.VMEM((_NB,), jnp.float32),
            pltpu.SemaphoreType.DMA,
            pltpu.SemaphoreType.DMA,
        ],
        compiler_params=pltpu.CompilerParams(needs_layout_passes=False),
    )
    def _sc_hist(neg_hbm, cnt_hbm, sum_hbm, chunk_a, chunk_b,
                 hcnt_v, hsum_v, mcnt_v, msum_v, sem_a, sem_b):
        wid = lax.axis_index("s") * 2 + lax.axis_index("c")
        lane_base = lax.iota(jnp.int32, 16) * _NB
        ones = jnp.ones((16,), jnp.float32)
        zeros16 = jnp.zeros((16,), jnp.float32)

        def zero_body(i, carry):
            for u in range(8):
                hcnt_v[pl.ds(i * 128 + u * 16, 16)] = zeros16
                hsum_v[pl.ds(i * 128 + u * 16, 16)] = zeros16
            return carry

        lax.fori_loop(0, _NB * _LN // 128, zero_body, 0)

        row0 = (wid >> 2) * _GROUP_ROWS
        col0 = (wid & 3) * _BAND
        bufs = (chunk_a, chunk_b)
        sems = (sem_a, sem_b)

        def start(ci):
            pltpu.async_copy(
                neg_hbm.at[pl.ds(row0 + ci * _UROWS, _UROWS),
                           pl.ds(col0, _BAND)],
                bufs[ci % 2], sems[ci % 2])

        start(0)
        for ci in range(_UNITS):
            pltpu.make_async_copy(
                neg_hbm.at[pl.ds(row0 + ci * _UROWS, _UROWS),
                           pl.ds(col0, _BAND)],
                bufs[ci % 2], sems[ci % 2]).wait()
            if ci + 1 < _UNITS:
                start(ci + 1)
            chunk_v = bufs[ci % 2]

            @plsc.parallel_loop(0, _UVEC, unroll=16)
            def _hist_body(i, chunk_v=chunk_v):
                r = i >> 4
                co = (i & 15) << 4
                v = chunk_v[r, pl.ds(co, 16)]
                bn = jnp.clip((v * _BIN_SCALE).astype(jnp.int32),
                              0, _NB - 1)
                idx = lane_base + bn
                plsc.addupdate_scatter(hcnt_v, [idx], ones)
                plsc.addupdate_scatter(hsum_v, [idx], v)

        def merge_body(i, carry):
            acc_c = zeros16
            acc_s = zeros16
            for l in range(_LN):
                acc_c = acc_c + hcnt_v[pl.ds(l * _NB + i * 16, 16)]
                acc_s = acc_s + hsum_v[pl.ds(l * _NB + i * 16, 16)]
            mcnt_v[pl.ds(i * 16, 16)] = acc_c
            msum_v[pl.ds(i * 16, 16)] = acc_s
            return carry

        lax.fori_loop(0, _NB // 16, merge_body, 0)

        pltpu.sync_copy(mcnt_v, cnt_hbm.at[wid])
        pltpu.sync_copy(msum_v, sum_hbm.at[wid])

    return _sc_hist


def _fin_body(hcnt_ref, hsum_ref, sum_ref, mm_ref, dice_ref, out_ref):
    l1_num = jnp.sum(sum_ref[0, :])
    l1_den = jnp.sum(sum_ref[1, :])
    pos_cnt = jnp.sum(sum_ref[2, :])
    neg_cnt_raw = jnp.sum(sum_ref[3, :])
    pos_loss_sum = jnp.sum(sum_ref[4, :])
    dmin = jnp.min(mm_ref[0, :])
    dmax = jnp.max(mm_ref[1, :])

    k_f = jnp.minimum(neg_cnt_raw, pos_cnt * _NEG_RATIO)
    k_i = k_f.astype(jnp.int32)
    k_if = k_i.astype(jnp.float32)

    cnt_tot = jnp.sum(hcnt_ref[...], axis=0)
    sum_tot = jnp.sum(hsum_ref[...], axis=0)
    bin2 = (lax.broadcasted_iota(jnp.int32, (_NBROW, _LANES), 0) * _LANES
            + lax.broadcasted_iota(jnp.int32, (_NBROW, _LANES), 1))

    def suffix(bidx):
        sel = bin2 >= bidx
        c = jnp.sum(jnp.where(sel, cnt_tot, 0.0))
        s = jnp.sum(jnp.where(sel, sum_tot, 0.0))
        return c, s

    def bin_bisect(_, carry):
        lo, hi = carry
        mid = (lo + hi) // 2
        c, _s = suffix(mid)
        pred = c >= k_if
        return (jnp.where(pred, mid, lo), jnp.where(pred, hi, mid))

    b_lo, _ = lax.fori_loop(0, 11, bin_bisect,
                            (jnp.int32(0), jnp.int32(_NB)))

    c_at, s_at = suffix(b_lo)
    c_above, s_above = suffix(b_lo + 1)
    c_in = c_at - c_above
    s_in = s_at - s_above
    take = k_if - c_above
    mean_in = s_in / jnp.maximum(c_in, 1.0)
    topk_sum = s_above + take * mean_in

    balanced = (pos_loss_sum + topk_sum) / (pos_cnt + k_f + _EPS)
    balanced = balanced * _BAL_SCALE

    a = 1.0 / (dmax - dmin)
    c0 = 1.0 - dmin * a
    dice_total = jnp.float32(0.0)
    for bb in range(_B):
        s_ptm = jnp.sum(dice_ref[0, bb, :])
        s_ptml = jnp.sum(dice_ref[1, bb, :])
        s_ppm = jnp.sum(dice_ref[2, bb, :])
        s_ppml = jnp.sum(dice_ref[3, bb, :])
        s_ttm = jnp.sum(dice_ref[4, bb, :])
        s_ttml = jnp.sum(dice_ref[5, bb, :])
        inter = a * s_ptml + c0 * s_ptm
        union = a * (s_ppml + s_ttml) + c0 * (s_ppm + s_ttm) + 2.0 * _DICE_EPS
        dice_total += 1.0 - 2.0 * inter / union
    dice = dice_total / _B

    l1 = jnp.where(l1_den > 0, l1_num / l1_den, jnp.float32(0.0)) * _L1_SCALE

    out_ref[0, 0] = l1 + balanced + dice


def _finalize(hcnt3, hsum3, sums, mm, dice):
    return pl.pallas_call(
        _fin_body,
        out_specs=pl.BlockSpec(memory_space=pltpu.SMEM),
        out_shape=jax.ShapeDtypeStruct((1, 1), jnp.float32),
    )(hcnt3, hsum3, sums, mm, dice)


def kernel(y_pr, y_gt):
    y_pr4 = y_pr.reshape(_B, 3, _ROWS_PER_B, _LANES)
    y_gt4 = y_gt.reshape(_B, 4, _ROWS_PER_B, _LANES)
    neg, sums, mm, dice = _stream(y_pr4, y_gt4)
    hcnt, hsum = _build_sc_hist()(neg.reshape(_ROWS, _LANES))
    out = _finalize(hcnt.reshape(_WRK, _NBROW, _LANES),
                    hsum.reshape(_WRK, _NBROW, _LANES), sums, mm, dice)
    return out[0, 0]


# speedup vs baseline: 1.6446x; 1.0090x over previous
import functools

import jax
import jax.numpy as jnp
from jax import lax
from jax.experimental import pallas as pl
from jax.experimental.pallas import tpu as pltpu
from jax.experimental.pallas import tpu_sc as plsc

_NEG_RATIO = 3.0
_BAL_SCALE = 5.0
_EPS = 1e-09
_L1_SCALE = 10.0
_DICE_EPS = 0.001

_B = 8
_ROWS_PER_B = 400
_LANES = 1024
_RB = 400
_JB = _ROWS_PER_B // _RB
_ROWS = _B * _ROWS_PER_B
_N = _ROWS * _LANES

_BIG = 3.0e38

_NB = 2048
_VMAX = 27.632
_BIN_SCALE = _NB / _VMAX
_BIN_W = _VMAX / _NB
_LN = 16
_WRK = 32
_NBROW = _NB // _LANES

_GROUP_ROWS = 400
_BAND = 256
_UROWS = 40
_UNITS = _GROUP_ROWS // _UROWS
_UVEC = _UROWS * _BAND // 16


def _stream_body(pr_ref, gt_ref, neg_ref, sum_ref, mm_ref, dice_ref):
    b = pl.program_id(0)
    j = pl.program_id(1)

    @pl.when(jnp.logical_and(b == 0, j == 0))
    def _init():
        sum_ref[...] = jnp.zeros((8, _LANES), jnp.float32)
        dice_ref[...] = jnp.zeros((8, _B, _LANES), jnp.float32)
        mm_ref[...] = jnp.zeros((8, _LANES), jnp.float32)
        mm_ref[0:1, :] = jnp.full((1, _LANES), _BIG, jnp.float32)
        mm_ref[1:2, :] = jnp.full((1, _LANES), -_BIG, jnp.float32)

    binary = pr_ref[0, 0]
    thresh_binary = pr_ref[0, 1]
    thresh = pr_ref[0, 2]
    gt = gt_ref[0, 0]
    mask = gt_ref[0, 1]
    thresh_map = gt_ref[0, 2]
    thresh_mask = gt_ref[0, 3]

    p = jnp.clip(binary, 1e-12, 1.0 - 1e-12)
    logp = jnp.clip(jnp.log(p), -100.0, None)
    log1mp = jnp.clip(jnp.log(1.0 - p), -100.0, None)
    loss = -(gt * logp + (1.0 - gt) * log1mp)

    pos_m = gt * mask
    neg_m = (1.0 - gt) * mask
    negl = loss * neg_m

    neg_ref[0] = negl

    def _acc_sum(row, val2d):
        sum_ref[row:row + 1, :] += jnp.sum(val2d, axis=0, keepdims=True)

    _acc_sum(0, jnp.abs(thresh - thresh_map) * thresh_mask)
    _acc_sum(1, thresh_mask)
    _acc_sum(2, pos_m)
    _acc_sum(3, neg_m)
    _acc_sum(4, loss * pos_m)

    mm_ref[0:1, :] = jnp.minimum(mm_ref[0:1, :],
                                 jnp.min(loss, axis=0, keepdims=True))
    mm_ref[1:2, :] = jnp.maximum(mm_ref[1:2, :],
                                 jnp.max(loss, axis=0, keepdims=True))

    ptm = thresh_binary * gt * mask
    ppm = thresh_binary * thresh_binary * mask
    ttm = gt * gt * mask

    def _acc_dice(q, val2d):
        dice_ref[q, pl.ds(b, 1), :] += jnp.sum(val2d, axis=0, keepdims=True)

    _acc_dice(0, ptm)
    _acc_dice(1, ptm * loss)
    _acc_dice(2, ppm)
    _acc_dice(3, ppm * loss)
    _acc_dice(4, ttm)
    _acc_dice(5, ttm * loss)


def _stream(y_pr4, y_gt4):
    return pl.pallas_call(
        _stream_body,
        grid=(_B, _JB),
        in_specs=[
            pl.BlockSpec((1, 3, _RB, _LANES), lambda b, j: (b, 0, j, 0)),
            pl.BlockSpec((1, 4, _RB, _LANES), lambda b, j: (b, 0, j, 0)),
        ],
        out_specs=[
            pl.BlockSpec((1, _RB, _LANES), lambda b, j: (b, j, 0)),
            pl.BlockSpec((8, _LANES), lambda b, j: (0, 0)),
            pl.BlockSpec((8, _LANES), lambda b, j: (0, 0)),
            pl.BlockSpec((8, _B, _LANES), lambda b, j: (0, 0, 0)),
        ],
        out_shape=[
            jax.ShapeDtypeStruct((_B, _ROWS_PER_B, _LANES), jnp.float32),
            jax.ShapeDtypeStruct((8, _LANES), jnp.float32),
            jax.ShapeDtypeStruct((8, _LANES), jnp.float32),
            jax.ShapeDtypeStruct((8, _B, _LANES), jnp.float32),
        ],
        compiler_params=pltpu.CompilerParams(
            dimension_semantics=("arbitrary", "arbitrary"),
        ),
    )(y_pr4, y_gt4)


@functools.lru_cache(maxsize=1)
def _build_sc_hist():
    @functools.partial(
        pl.kernel,
        out_type=(
            jax.ShapeDtypeStruct((_WRK, _NB), jnp.float32),
            jax.ShapeDtypeStruct((_WRK, _NB), jnp.float32),
        ),
        mesh=plsc.VectorSubcoreMesh(core_axis_name="c", subcore_axis_name="s"),
        scratch_types=[
            pltpu.VMEM((_UROWS, _BAND), jnp.float32),
            pltpu.VMEM((_UROWS, _BAND), jnp.float32),
            pltpu.VMEM((_NB * _LN,), jnp.float32),
            pltpu.VMEM((_NB * _LN,), jnp.float32),
            pltpu.VMEM((_NB,), jnp.float32),
            pltpu.VMEM((_NB,), jnp.float32),
            pltpu.SemaphoreType.DMA,
            pltpu.SemaphoreType.DMA,
        ],
        compiler_params=pltpu.CompilerParams(needs_layout_passes=False),
    )
    def _sc_hist(neg_hbm, cnt_hbm, sum_hbm, chunk_a, chunk_b,
                 hcnt_v, hsum_v, mcnt_v, msum_v, sem_a, sem_b):
        wid = lax.axis_index("s") * 2 + lax.axis_index("c")
        lane_base = lax.iota(jnp.int32, 16) * _NB
        ones = jnp.ones((16,), jnp.float32)
        zeros16 = jnp.zeros((16,), jnp.float32)

        def zero_body(i, carry):
            for u in range(8):
                hcnt_v[pl.ds(i * 128 + u * 16, 16)] = zeros16
                hsum_v[pl.ds(i * 128 + u * 16, 16)] = zeros16
            return carry

        lax.fori_loop(0, _NB * _LN // 128, zero_body, 0)

        row0 = (wid >> 2) * _GROUP_ROWS
        col0 = (wid & 3) * _BAND
        bufs = (chunk_a, chunk_b)
        sems = (sem_a, sem_b)

        def start(ci):
            pltpu.async_copy(
                neg_hbm.at[pl.ds(row0 + ci * _UROWS, _UROWS),
                           pl.ds(col0, _BAND)],
                bufs[ci % 2], sems[ci % 2])

        start(0)
        for ci in range(_UNITS):
            pltpu.make_async_copy(
                neg_hbm.at[pl.ds(row0 + ci * _UROWS, _UROWS),
                           pl.ds(col0, _BAND)],
                bufs[ci % 2], sems[ci % 2]).wait()
            if ci + 1 < _UNITS:
                start(ci + 1)
            chunk_v = bufs[ci % 2]

            @plsc.parallel_loop(0, _UVEC, unroll=16)
            def _hist_body(i, chunk_v=chunk_v):
                r = i >> 4
                co = (i & 15) << 4
                v = chunk_v[r, pl.ds(co, 16)]
                bn = jnp.clip((v * _BIN_SCALE).astype(jnp.int32),
                              0, _NB - 1)
                idx = lane_base + bn
                plsc.addupdate_scatter(hcnt_v, [idx], ones)
                plsc.addupdate_scatter(hsum_v, [idx], v)

        def merge_body(i, carry):
            acc_c = zeros16
            acc_s = zeros16
            for l in range(_LN):
                acc_c = acc_c + hcnt_v[pl.ds(l * _NB + i * 16, 16)]
                acc_s = acc_s + hsum_v[pl.ds(l * _NB + i * 16, 16)]
            mcnt_v[pl.ds(i * 16, 16)] = acc_c
            msum_v[pl.ds(i * 16, 16)] = acc_s
            return carry

        lax.fori_loop(0, _NB // 16, merge_body, 0)

        pltpu.sync_copy(mcnt_v, cnt_hbm.at[wid])
        pltpu.sync_copy(msum_v, sum_hbm.at[wid])

    return _sc_hist


def _fin_body(hcnt_ref, hsum_ref, sum_ref, mm_ref, dice_ref, out_ref):
    l1_num = jnp.sum(sum_ref[0, :])
    l1_den = jnp.sum(sum_ref[1, :])
    pos_cnt = jnp.sum(sum_ref[2, :])
    neg_cnt_raw = jnp.sum(sum_ref[3, :])
    pos_loss_sum = jnp.sum(sum_ref[4, :])
    dmin = jnp.min(mm_ref[0, :])
    dmax = jnp.max(mm_ref[1, :])

    k_f = jnp.minimum(neg_cnt_raw, pos_cnt * _NEG_RATIO)
    k_i = k_f.astype(jnp.int32)
    k_if = k_i.astype(jnp.float32)

    cnt_tot = jnp.sum(hcnt_ref[...], axis=0)
    sum_tot = jnp.sum(hsum_ref[...], axis=0)
    bin2 = (lax.broadcasted_iota(jnp.int32, (_NBROW, _LANES), 0) * _LANES
            + lax.broadcasted_iota(jnp.int32, (_NBROW, _LANES), 1))

    def suffix(bidx):
        sel = bin2 >= bidx
        c = jnp.sum(jnp.where(sel, cnt_tot, 0.0))
        s = jnp.sum(jnp.where(sel, sum_tot, 0.0))
        return c, s

    def bin_bisect(_, carry):
        lo, hi = carry
        mid = (lo + hi) // 2
        c, _s = suffix(mid)
        pred = c >= k_if
        return (jnp.where(pred, mid, lo), jnp.where(pred, hi, mid))

    b_lo, _ = lax.fori_loop(0, 11, bin_bisect,
                            (jnp.int32(0), jnp.int32(_NB)))

    c_at, s_at = suffix(b_lo)
    c_above, s_above = suffix(b_lo + 1)
    c_in = c_at - c_above
    s_in = s_at - s_above
    take = k_if - c_above
    mean_in = s_in / jnp.maximum(c_in, 1.0)
    topk_sum = s_above + take * mean_in

    balanced = (pos_loss_sum + topk_sum) / (pos_cnt + k_f + _EPS)
    balanced = balanced * _BAL_SCALE

    a = 1.0 / (dmax - dmin)
    c0 = 1.0 - dmin * a
    dice_total = jnp.float32(0.0)
    for bb in range(_B):
        s_ptm = jnp.sum(dice_ref[0, bb, :])
        s_ptml = jnp.sum(dice_ref[1, bb, :])
        s_ppm = jnp.sum(dice_ref[2, bb, :])
        s_ppml = jnp.sum(dice_ref[3, bb, :])
        s_ttm = jnp.sum(dice_ref[4, bb, :])
        s_ttml = jnp.sum(dice_ref[5, bb, :])
        inter = a * s_ptml + c0 * s_ptm
        union = a * (s_ppml + s_ttml) + c0 * (s_ppm + s_ttm) + 2.0 * _DICE_EPS
        dice_total += 1.0 - 2.0 * inter / union
    dice = dice_total / _B

    l1 = jnp.where(l1_den > 0, l1_num / l1_den, jnp.float32(0.0)) * _L1_SCALE

    out_ref[0, 0] = l1 + balanced + dice


def _finalize(hcnt3, hsum3, sums, mm, dice):
    return pl.pallas_call(
        _fin_body,
        out_specs=pl.BlockSpec(memory_space=pltpu.SMEM),
        out_shape=jax.ShapeDtypeStruct((1, 1), jnp.float32),
    )(hcnt3, hsum3, sums, mm, dice)


def kernel(y_pr, y_gt):
    y_pr4 = y_pr.reshape(_B, 3, _ROWS_PER_B, _LANES)
    y_gt4 = y_gt.reshape(_B, 4, _ROWS_PER_B, _LANES)
    neg, sums, mm, dice = _stream(y_pr4, y_gt4)
    hcnt, hsum = _build_sc_hist()(neg.reshape(_ROWS, _LANES))
    out = _finalize(hcnt.reshape(_WRK, _NBROW, _LANES),
                    hsum.reshape(_WRK, _NBROW, _LANES), sums, mm, dice)
    return out[0, 0]
